# baseline (speedup 1.0000x reference)
import functools

import jax
import jax.numpy as jnp
from jax import lax
from jax.experimental import pallas as pl
from jax.experimental.pallas import tpu as pltpu
from jax.experimental.pallas import tpu_sc as plsc

N_NODES = 10000
N_EDGES = 320000
D = 128

NC = 2
NS = 16
NW = NC * NS
E_PER_TILE = N_EDGES // NW
CHUNK = 112
N_CHUNKS = 90
E_PAD_TILE = CHUNK * N_CHUNKS
N_PAD = 10112
ROWS_PER_TILE = N_PAD // NS
DUMP_ROW = N_NODES + 64
NB = 3
NI = 6


def _sc_scatter(x, cw, rowp, zeros):
    mesh = plsc.VectorSubcoreMesh(core_axis_name="c", subcore_axis_name="s")

    @functools.partial(
        pl.kernel,
        mesh=mesh,
        out_type=jax.ShapeDtypeStruct((NC, N_PAD, D), jnp.float32),
        scratch_types=[
            pltpu.VMEM_SHARED((N_PAD, D), jnp.float32),
            [pltpu.VMEM((2 * CHUNK,), jnp.int32) for _ in range(NI)],
            [pltpu.VMEM((CHUNK,), jnp.int32) for _ in range(NB)],
            [pltpu.VMEM((CHUNK, D), jnp.float32) for _ in range(NB)],
            [pltpu.SemaphoreType.DMA for _ in range(NI)],
            [pltpu.SemaphoreType.DMA for _ in range(NB)],
            [pltpu.SemaphoreType.DMA for _ in range(NB)],
        ],
    )
    def k(x_hbm, cw_hbm, row_hbm, z_hbm, out_hbm,
          acc, ibuf, rowb, gbufs, si, sr, sg):
        cid = lax.axis_index("c")
        sid = lax.axis_index("s")
        wid = cid * NS + sid

        rbase = sid * ROWS_PER_TILE
        pltpu.sync_copy(z_hbm.at[pl.ds(rbase, ROWS_PER_TILE)],
                        acc.at[pl.ds(rbase, ROWS_PER_TILE)])
        plsc.subcore_barrier()

        def cw_off(i):
            return pl.multiple_of((wid * N_CHUNKS + i) * 2 * CHUNK, 8)

        def row_off(i):
            return pl.multiple_of(wid * E_PAD_TILE + i * CHUNK, 8)

        def issue_fetch(i, p):
            pltpu.async_copy(cw_hbm.at[pl.ds(cw_off(i), 2 * CHUNK)],
                             ibuf[p], si[p])

        def issue_row(i, p):
            pltpu.async_copy(row_hbm.at[pl.ds(row_off(i), CHUNK)],
                             rowb[p], sr[p])

        def issue_gather(i, pi, pg):
            pltpu.make_async_copy(cw_hbm.at[pl.ds(cw_off(i), 2 * CHUNK)],
                                  ibuf[pi], si[pi]).wait()
            pltpu.async_copy(x_hbm.at[ibuf[pi].at[pl.ds(0, CHUNK)]],
                             gbufs[pg], sg[pg])

        def scale(i, p3, p6):
            def g_body(g, c):
                w16 = lax.bitcast_convert_type(
                    ibuf[p6][pl.ds(CHUNK + g * 16, 16)], jnp.float32)
                for t in range(16):
                    w = w16[t]
                    for j in range(D // 16):
                        sl = pl.ds(j * 16, 16)
                        gbufs[p3][g * 16 + t, sl] = (
                            gbufs[p3][g * 16 + t, sl] * w)
                return c

            lax.fori_loop(0, CHUNK // 16, g_body, 0)

        def slot(i, d):
            p3, p6 = d % NB, d % NI

            @pl.when(i + 5 < N_CHUNKS)
            def _():
                issue_fetch(i + 5, (p6 + 5) % NI)

            @pl.when(i + 2 < N_CHUNKS)
            def _():
                issue_gather(i + 2, (p6 + 2) % NI, (p3 + 2) % NB)

            pltpu.make_async_copy(x_hbm.at[ibuf[p6].at[pl.ds(0, CHUNK)]],
                                  gbufs[p3], sg[p3]).wait()
            scale(i, p3, p6)
            pltpu.make_async_copy(row_hbm.at[pl.ds(row_off(i), CHUNK)],
                                  rowb[p3], sr[p3]).wait()
            pltpu.sync_copy(gbufs[p3], acc.at[rowb[p3]], add=True)

            @pl.when(i + 3 < N_CHUNKS)
            def _():
                issue_row(i + 3, p3)

        for i2 in range(5):
            issue_fetch(i2, i2)
        for i2 in range(2):
            issue_gather(i2, i2, i2)
        for i2 in range(NB):
            issue_row(i2, i2)

        def outer(g, carry):
            for d in range(NI):
                slot(g * NI + d, d)
            return carry

        lax.fori_loop(0, N_CHUNKS // NI, outer, 0)

        plsc.subcore_barrier()
        pltpu.sync_copy(acc.at[pl.ds(rbase, ROWS_PER_TILE)],
                        out_hbm.at[cid, pl.ds(rbase, ROWS_PER_TILE)])

    return k(x, cw, rowp, zeros)


def _tc_finish(partials, W):
    ROWS_BLK = 2000

    def body(p_ref, w_ref, o_ref):
        o_ref[...] = jnp.dot(p_ref[0] + p_ref[1], w_ref[...],
                             preferred_element_type=jnp.float32)

    return pl.pallas_call(
        body,
        grid=(N_NODES // ROWS_BLK,),
        in_specs=[
            pl.BlockSpec((NC, ROWS_BLK, D), lambda i: (0, i, 0)),
            pl.BlockSpec((D, D), lambda i: (0, 0)),
        ],
        out_specs=pl.BlockSpec((ROWS_BLK, D), lambda i: (i, 0)),
        out_shape=jax.ShapeDtypeStruct((N_NODES, D), jnp.float32),
    )(partials, W)


@jax.jit
def kernel(x, edge_index, edge_weight, W):
    pad = E_PAD_TILE - E_PER_TILE
    row = edge_index[0].astype(jnp.int32).reshape(NW, E_PER_TILE)
    col = edge_index[1].astype(jnp.int32).reshape(NW, E_PER_TILE)
    ewb = jax.lax.bitcast_convert_type(
        edge_weight.astype(jnp.float32), jnp.int32).reshape(NW, E_PER_TILE)
    rowp = jnp.pad(row, ((0, 0), (0, pad)),
                   constant_values=DUMP_ROW).reshape(NW * E_PAD_TILE)
    colp = jnp.pad(col, ((0, 0), (0, pad)))
    ewp = jnp.pad(ewb, ((0, 0), (0, pad)))
    cw = jnp.stack([colp.reshape(NW, N_CHUNKS, CHUNK),
                    ewp.reshape(NW, N_CHUNKS, CHUNK)],
                   axis=2).reshape(NW * N_CHUNKS * 2 * CHUNK)
    zeros = jnp.zeros((N_PAD, D), jnp.float32)
    partials = _sc_scatter(x, cw, rowp, zeros)
    return _tc_finish(partials, W)


# speedup vs baseline: 1.6445x; 1.6445x over previous
import functools

import jax
import jax.numpy as jnp
from jax import lax
from jax.experimental import pallas as pl
from jax.experimental.pallas import tpu as pltpu
from jax.experimental.pallas import tpu_sc as plsc

N_NODES = 10000
N_EDGES = 320000
D = 128

NC = 2
NS = 16
NW = NC * NS
E_PER_TILE = N_EDGES // NW
CHUNK = 80
N_CHUNKS = E_PER_TILE // CHUNK
NB = 4
NCOL = 8
N_PAD = 10112
ROWS_PER_TILE = N_PAD // NS


def _sc_scatter(x, ei, ew, zeros):
    mesh = plsc.VectorSubcoreMesh(core_axis_name="c", subcore_axis_name="s")

    @functools.partial(
        pl.kernel,
        mesh=mesh,
        out_type=jax.ShapeDtypeStruct((NC, N_PAD, D), jnp.float32),
        scratch_types=[
            pltpu.VMEM_SHARED((N_PAD, D), jnp.float32),
            [pltpu.VMEM((CHUNK,), jnp.int32) for _ in range(NCOL)],
            [pltpu.VMEM((CHUNK,), jnp.int32) for _ in range(NB)],
            [pltpu.VMEM((CHUNK,), jnp.float32) for _ in range(NB)],
            [pltpu.VMEM((CHUNK, D), jnp.float32) for _ in range(NB)],
            [pltpu.SemaphoreType.DMA for _ in range(NCOL)],
            [pltpu.SemaphoreType.DMA for _ in range(NB)],
            [pltpu.SemaphoreType.DMA for _ in range(NB)],
            [pltpu.SemaphoreType.DMA for _ in range(NB)],
        ],
    )
    def k(x_hbm, ei_hbm, ew_hbm, z_hbm, out_hbm,
          acc, colb, rowb, ewb, gbufs, scol, sr, se, sg):
        cid = lax.axis_index("c")
        sid = lax.axis_index("s")
        wid = cid * NS + sid

        rbase = sid * ROWS_PER_TILE
        pltpu.sync_copy(z_hbm.at[pl.ds(rbase, ROWS_PER_TILE)],
                        acc.at[pl.ds(rbase, ROWS_PER_TILE)])
        plsc.subcore_barrier()

        ebase = wid * E_PER_TILE

        def row_off(i):
            return pl.multiple_of(ebase + i * CHUNK, 8)

        def col_off(i):
            return pl.multiple_of(N_EDGES + ebase + i * CHUNK, 8)

        def issue_col(i, p):
            pltpu.async_copy(ei_hbm.at[pl.ds(col_off(i), CHUNK)],
                             colb[p], scol[p])

        def issue_row_ew(i, p):
            pltpu.async_copy(ei_hbm.at[pl.ds(row_off(i), CHUNK)],
                             rowb[p], sr[p])
            pltpu.async_copy(ew_hbm.at[pl.ds(row_off(i), CHUNK)],
                             ewb[p], se[p])

        def issue_gather(i, pc, pg):
            pltpu.make_async_copy(ei_hbm.at[pl.ds(col_off(i), CHUNK)],
                                  colb[pc], scol[pc]).wait()
            pltpu.async_copy(x_hbm.at[colb[pc]], gbufs[pg], sg[pg])

        def scale(i, p):
            def g_body(g, c):
                w16 = ewb[p][pl.ds(g * 16, 16)]
                for t in range(16):
                    w = w16[t]
                    for j in range(D // 16):
                        sl = pl.ds(j * 16, 16)
                        gbufs[p][g * 16 + t, sl] = gbufs[p][g * 16 + t, sl] * w
                return c

            lax.fori_loop(0, CHUNK // 16, g_body, 0)

        def slot(i, p, pc, do_col=True, do_gather=True, do_row_ew=True):
            if do_col:
                issue_col(i + 5, (pc + 5) % NCOL)
            if do_gather:
                issue_gather(i + 3, (pc + 3) % NCOL, (p + 3) % NB)
            pltpu.make_async_copy(ew_hbm.at[pl.ds(row_off(i), CHUNK)],
                                  ewb[p], se[p]).wait()
            pltpu.make_async_copy(x_hbm.at[colb[pc]], gbufs[p], sg[p]).wait()
            scale(i, p)
            pltpu.make_async_copy(ei_hbm.at[pl.ds(row_off(i), CHUNK)],
                                  rowb[p], sr[p]).wait()
            pltpu.sync_copy(gbufs[p], acc.at[rowb[p]], add=True)
            if do_row_ew:
                issue_row_ew(i + 4, p)

        for i2 in range(5):
            issue_col(i2, i2)
        for i2 in range(3):
            issue_gather(i2, i2, i2)
        for i2 in range(NB):
            issue_row_ew(i2, i2)

        def outer(g, carry):
            for d in range(8):
                slot(g * 8 + d, d % NB, d % NCOL)
            return carry

        lax.fori_loop(0, 15, outer, 0)
        slot(120, 0, 0, do_col=False)
        slot(121, 1, 1, do_col=False, do_row_ew=False)
        slot(122, 2, 2, do_col=False, do_gather=False, do_row_ew=False)
        slot(123, 3, 3, do_col=False, do_gather=False, do_row_ew=False)
        slot(124, 0, 4, do_col=False, do_gather=False, do_row_ew=False)

        plsc.subcore_barrier()
        pltpu.sync_copy(acc.at[pl.ds(rbase, ROWS_PER_TILE)],
                        out_hbm.at[cid, pl.ds(rbase, ROWS_PER_TILE)])

    return k(x, ei, ew, zeros)


def _tc_finish(partials, W):
    ROWS_BLK = 2000

    def body(p_ref, w_ref, o_ref):
        o_ref[...] = jnp.dot(p_ref[0] + p_ref[1], w_ref[...],
                             preferred_element_type=jnp.float32)

    return pl.pallas_call(
        body,
        grid=(N_NODES // ROWS_BLK,),
        in_specs=[
            pl.BlockSpec((NC, ROWS_BLK, D), lambda i: (0, i, 0)),
            pl.BlockSpec((D, D), lambda i: (0, 0)),
        ],
        out_specs=pl.BlockSpec((ROWS_BLK, D), lambda i: (i, 0)),
        out_shape=jax.ShapeDtypeStruct((N_NODES, D), jnp.float32),
    )(partials, W)


@jax.jit
def kernel(x, edge_index, edge_weight, W):
    ei = edge_index.astype(jnp.int32).reshape(2 * N_EDGES)
    ew = edge_weight.astype(jnp.float32)
    zeros = jnp.zeros((N_PAD, D), jnp.float32)
    partials = _sc_scatter(x, ei, ew, zeros)
    return _tc_finish(partials, W)
